# Initial kernel scaffold; baseline (speedup 1.0000x reference)
import jax
import jax.numpy as jnp
from jax.experimental import pallas as pl


def kernel(weather, uppers, ids):
    raise NotImplementedError("write your pallas kernel here")


# speedup vs baseline: 91.8039x; 91.8039x over previous
import functools

import jax
import jax.numpy as jnp
import numpy as np
from jax import lax
from jax.experimental import pallas as pl
from jax.experimental.pallas import tpu as pltpu
from jax.experimental.pallas import tpu_sc as plsc

_B, _S, _V = 4096, 2048, 3
_N = 256
_UNK = 1
_ROW = _S * _V
_NW = 32
_ROWS_PER = _B // _NW
_BLKS = _ROW // 48

_LANES = np.arange(16)
_VPH = np.stack([(16 * p + _LANES) % _V for p in range(3)])
_D0 = np.stack([_VPH[p] * _S + (16 * p + _LANES) // _V for p in range(3)])


def _tok_body(w_hbm, up_hbm, ids_hbm, pf_hbm, pi_hbm, out_hbm,
              up_v, ids_v, pf_v, pi_v, in_v, out_v):
    wid = lax.axis_index("s") * 2 + lax.axis_index("c")
    pltpu.sync_copy(up_hbm, up_v)
    pltpu.sync_copy(ids_hbm, ids_v)
    pltpu.sync_copy(pf_hbm, pf_v)
    pltpu.sync_copy(pi_hbm, pi_v)

    inv_p = [pf_v[pl.ds(p * 16, 16)] for p in range(3)]
    c_p = [pf_v[pl.ds(48 + p * 16, 16)] for p in range(3)]
    voff_p = [pi_v[pl.ds(p * 16, 16)] for p in range(3)]
    d0_p = [pi_v[pl.ds(48 + p * 16, 16)] for p in range(3)]

    base_row = wid * _ROWS_PER

    def row_body(r, carry):
        off = (base_row + r) * _ROW
        pltpu.sync_copy(w_hbm.at[pl.ds(off, _ROW)], in_v)

        def blk(t, c2):
            toff = t * 48
            for p in range(3):
                x = in_v[pl.ds(toff + p * 16, 16)]
                tf = x * inv_p[p] + c_p[p]
                tf = jnp.minimum(jnp.maximum(tf, 0.0), float(_N))
                g = tf.astype(jnp.int32)
                ia = jnp.maximum(g - 1, 0) + voff_p[p]
                ib = jnp.minimum(g, _N - 1) + voff_p[p]
                a = plsc.load_gather(up_v, [ia])
                b = plsc.load_gather(up_v, [ib])
                i = (g
                     + jnp.where((b <= x) & (g < _N), 1, 0)
                     - jnp.where((a > x) & (g > 0), 1, 0))
                ii = jnp.minimum(i, _N - 1) + voff_p[p]
                tok = plsc.load_gather(ids_v, [ii])
                tok = jnp.where(i == _N, _UNK, tok)
                plsc.store_scatter(out_v, [d0_p[p] + t * 16], tok)
            return c2

        lax.fori_loop(0, _BLKS, blk, 0)
        pltpu.sync_copy(out_v, out_hbm.at[pl.ds(off, _ROW)])
        return carry

    lax.fori_loop(0, _ROWS_PER, row_body, 0)


@functools.partial(
    pl.kernel,
    mesh=plsc.VectorSubcoreMesh(core_axis_name="c", subcore_axis_name="s"),
    compiler_params=pltpu.CompilerParams(needs_layout_passes=False),
    out_type=jax.ShapeDtypeStruct((_B * _ROW,), jnp.int32),
    scratch_types=[
        pltpu.VMEM((_V * _N,), jnp.float32),
        pltpu.VMEM((_V * _N,), jnp.int32),
        pltpu.VMEM((96,), jnp.float32),
        pltpu.VMEM((96,), jnp.int32),
        pltpu.VMEM((_ROW,), jnp.float32),
        pltpu.VMEM((_ROW,), jnp.int32),
    ],
)
def _tokenize(*args):
    _tok_body(*args)


def kernel(weather, uppers, ids):
    lo = uppers[:, 0]
    inv = ((_N - 1) / (uppers[:, -1] - lo)).astype(jnp.float32)
    inv_l = inv[_VPH]
    c_l = (jnp.float32(1.0) - lo[_VPH] * inv_l).astype(jnp.float32)
    pf = jnp.concatenate([inv_l.reshape(-1), c_l.reshape(-1)])
    pi = jnp.concatenate([
        jnp.asarray(_VPH * _N, jnp.int32).reshape(-1),
        jnp.asarray(_D0, jnp.int32).reshape(-1),
    ])
    tok = _tokenize(
        weather.reshape(-1),
        uppers.reshape(-1),
        ids.reshape(-1).astype(jnp.int32),
        pf,
        pi,
    )
    tok_ids = tok.reshape(_B, _ROW)
    input_mask = jnp.zeros((_B, _ROW), bool)
    target_mask = jnp.ones((_B, _ROW), bool)
    decoder_attention_mask = jnp.zeros((_B, _ROW), bool)
    return tok_ids, input_mask, target_mask, decoder_attention_mask


# speedup vs baseline: 1730.6671x; 18.8518x over previous
import functools

import jax
import jax.numpy as jnp
from jax import lax
from jax.experimental import pallas as pl
from jax.experimental.pallas import tpu as pltpu
from jax.experimental.pallas import tpu_sc as plsc

_B, _S, _V = 4096, 2048, 3
_N = 256
_UNK = 1
_NW = 32
_TR = _B // 8
_TR_PER = _TR // _NW
_CHUNK = 8 * _S
_VECS = _CHUNK // 16


def _tok_body(w_hbm, up_hbm, ids_hbm, pf_hbm, out_hbm, up_v, ids_v, pf_v,
              in_v, out_v):
    wid = lax.axis_index("s") * 2 + lax.axis_index("c")
    pltpu.sync_copy(up_hbm, up_v)
    pltpu.sync_copy(ids_hbm, ids_v)
    pltpu.sync_copy(pf_hbm, pf_v)

    r0 = wid * _TR_PER
    for v in range(_V):
        inv_b = pf_v[pl.ds(v * 16, 16)]
        c_b = pf_v[pl.ds(48 + v * 16, 16)]
        voff = v * _N

        def row_body(j, carry, v=v, inv_b=inv_b, c_b=c_b, voff=voff):
            row8 = (r0 + j) * 8
            pltpu.sync_copy(w_hbm.at[v, pl.ds(row8, 8), :], in_v)

            for r in range(8):
                def vec_body(i, c2, r=r):
                    o = i * 16
                    x = in_v[r, pl.ds(o, 16)]
                    tf = x * inv_b + c_b
                    tf = jnp.minimum(jnp.maximum(tf, 0.0), float(_N))
                    g = tf.astype(jnp.int32)
                    ia = jnp.maximum(g - 1, 0) + voff
                    ib = jnp.minimum(g, _N - 1) + voff
                    a = plsc.load_gather(up_v, [ia])
                    b = plsc.load_gather(up_v, [ib])
                    i2 = (g
                          + jnp.where((b <= x) & (g < _N), 1, 0)
                          - jnp.where((a > x) & (g > 0), 1, 0))
                    ii = jnp.minimum(i2, _N - 1) + voff
                    tok = plsc.load_gather(ids_v, [ii])
                    tok = jnp.where(i2 == _N, _UNK, tok)
                    out_v[r, pl.ds(o, 16)] = tok
                    return c2

                lax.fori_loop(0, _S // 16, vec_body, 0, unroll=8)
            pltpu.sync_copy(out_v, out_hbm.at[pl.ds(row8, 8), pl.ds(v * _S, _S)])
            return carry

        lax.fori_loop(0, _TR_PER, row_body, 0)


@functools.partial(
    pl.kernel,
    mesh=plsc.VectorSubcoreMesh(core_axis_name="c", subcore_axis_name="s"),
    compiler_params=pltpu.CompilerParams(needs_layout_passes=False),
    out_type=jax.ShapeDtypeStruct((_B, _V * _S), jnp.int32),
    scratch_types=[
        pltpu.VMEM((_V * _N,), jnp.float32),
        pltpu.VMEM((_V * _N,), jnp.int32),
        pltpu.VMEM((96,), jnp.float32),
        pltpu.VMEM((8, _S), jnp.float32),
        pltpu.VMEM((8, _S), jnp.int32),
    ],
)
def _tokenize(*args):
    _tok_body(*args)


def kernel(weather, uppers, ids):
    lo = uppers[:, 0]
    inv = ((_N - 1) / (uppers[:, -1] - lo)).astype(jnp.float32)
    inv_l = jnp.repeat(inv, 16)
    c_l = jnp.repeat(jnp.float32(1.0) - lo * inv, 16)
    pf = jnp.concatenate([inv_l, c_l])
    tok_ids = _tokenize(
        weather.transpose(2, 0, 1),
        uppers.reshape(-1),
        ids.reshape(-1).astype(jnp.int32),
        pf,
    )
    input_mask = jnp.zeros((_B, _V * _S), bool)
    target_mask = jnp.ones((_B, _V * _S), bool)
    decoder_attention_mask = jnp.zeros((_B, _V * _S), bool)
    return tok_ids, input_mask, target_mask, decoder_attention_mask


# speedup vs baseline: 5481.5017x; 3.1673x over previous
import functools

import jax
import jax.numpy as jnp
from jax import lax
from jax.experimental import pallas as pl
from jax.experimental.pallas import tpu as pltpu
from jax.experimental.pallas import tpu_sc as plsc

_B, _S, _V = 4096, 2048, 3
_N = 256
_UNK = 1
_NW = 32
_TR = _B // 8
_TR_PER = _TR // _NW
_CHUNK = 8 * _S
_VECS = _CHUNK // 16


def _tok_body(w_hbm, up_hbm, ids_hbm, pf_hbm, out_hbm, up_v, ids_v, pf_v,
              in_v, out_v):
    wid = lax.axis_index("s") * 2 + lax.axis_index("c")
    pltpu.sync_copy(up_hbm, up_v)
    pltpu.sync_copy(ids_hbm, ids_v)
    pltpu.sync_copy(pf_hbm, pf_v)

    r0 = wid * _TR_PER
    for v in range(_V):
        inv_b = pf_v[pl.ds(v * 16, 16)]
        c_b = pf_v[pl.ds(48 + v * 16, 16)]
        voff = v * _N

        def row_body(j, carry, v=v, inv_b=inv_b, c_b=c_b, voff=voff):
            row8 = (r0 + j) * 8
            pltpu.sync_copy(w_hbm.at[v, pl.ds(row8, 8), :], in_v)

            for r in range(8):
                @plsc.parallel_loop(0, _S // 16, unroll=8)
                def vec_body(i, r=r):
                    o = i * 16
                    x = in_v[r, pl.ds(o, 16)]
                    tf = x * inv_b + c_b
                    tf = jnp.minimum(jnp.maximum(tf, 0.0), float(_N))
                    g = tf.astype(jnp.int32)
                    ia = jnp.maximum(g - 1, 0) + voff
                    ib = jnp.minimum(g, _N - 1) + voff
                    a = plsc.load_gather(up_v, [ia])
                    b = plsc.load_gather(up_v, [ib])
                    i2 = (g
                          + jnp.where((b <= x) & (g < _N), 1, 0)
                          - jnp.where((a > x) & (g > 0), 1, 0))
                    ii = jnp.minimum(i2, _N - 1) + voff
                    tok = plsc.load_gather(ids_v, [ii])
                    tok = jnp.where(i2 == _N, _UNK, tok)
                    out_v[r, pl.ds(o, 16)] = tok
            pltpu.sync_copy(out_v, out_hbm.at[pl.ds(row8, 8), pl.ds(v * _S, _S)])
            return carry

        lax.fori_loop(0, _TR_PER, row_body, 0)


@functools.partial(
    pl.kernel,
    mesh=plsc.VectorSubcoreMesh(core_axis_name="c", subcore_axis_name="s"),
    compiler_params=pltpu.CompilerParams(needs_layout_passes=False),
    out_type=jax.ShapeDtypeStruct((_B, _V * _S), jnp.int32),
    scratch_types=[
        pltpu.VMEM((_V * _N,), jnp.float32),
        pltpu.VMEM((_V * _N,), jnp.int32),
        pltpu.VMEM((96,), jnp.float32),
        pltpu.VMEM((8, _S), jnp.float32),
        pltpu.VMEM((8, _S), jnp.int32),
    ],
)
def _tokenize(*args):
    _tok_body(*args)


def kernel(weather, uppers, ids):
    lo = uppers[:, 0]
    inv = ((_N - 1) / (uppers[:, -1] - lo)).astype(jnp.float32)
    inv_l = jnp.repeat(inv, 16)
    c_l = jnp.repeat(jnp.float32(1.0) - lo * inv, 16)
    pf = jnp.concatenate([inv_l, c_l])
    tok_ids = _tokenize(
        weather.transpose(2, 0, 1),
        uppers.reshape(-1),
        ids.reshape(-1).astype(jnp.int32),
        pf,
    )
    input_mask = jnp.zeros((_B, _V * _S), bool)
    target_mask = jnp.ones((_B, _V * _S), bool)
    decoder_attention_mask = jnp.zeros((_B, _V * _S), bool)
    return tok_ids, input_mask, target_mask, decoder_attention_mask
